# baseline (speedup 1.0000x reference)
import functools

import jax
import jax.numpy as jnp
from jax import lax
from jax.experimental import pallas as pl
from jax.experimental.pallas import tpu as pltpu
from jax.experimental.pallas import tpu_sc as plsc

LOSS_WEIGHT = 0.005
THRESH_SCALE = 0.05

NC = 2
NS = 16
L = 16
NW = NC * NS
CH = 64

_GDN = lax.GatherDimensionNumbers(
    offset_dims=(), collapsed_slice_dims=(0,), start_index_map=(0,)
)


def _shuffle(v, perm):
    return lax.gather(
        v,
        perm[:, None],
        _GDN,
        (1,),
        mode=lax.GatherScatterMode.PROMISE_IN_BOUNDS,
    )


def _make_kernel(B, D, C):
    bpw = B // NW
    nch = bpw // CH
    ngrp = CH // L

    mesh = plsc.VectorSubcoreMesh(core_axis_name="c", subcore_axis_name="s")

    @functools.partial(
        pl.kernel,
        mesh=mesh,
        out_type=jax.ShapeDtypeStruct((NW, L), jnp.float32),
        scratch_types=[
            pltpu.VMEM((nch, CH), jnp.int32),
            pltpu.VMEM((CH, D), jnp.float32),
            pltpu.VMEM((CH, D), jnp.float32),
            pltpu.VMEM((CH, D), jnp.float32),
            pltpu.VMEM((CH, D), jnp.float32),
            pltpu.VMEM((CH, D), jnp.float32),
            pltpu.VMEM((CH, D), jnp.float32),
            pltpu.VMEM((CH,), jnp.float32),
            pltpu.VMEM((CH,), jnp.float32),
            pltpu.VMEM((CH,), jnp.float32),
            pltpu.VMEM((L,), jnp.float32),
            pltpu.SemaphoreType.DMA,
            pltpu.SemaphoreType.DMA,
            pltpu.SemaphoreType.DMA,
        ],
    )
    def k(x_hbm, y_hbm, w_hbm, g_hbm, out_hbm, idx_v,
          x_v0, x_v1, x_v2, w_v0, w_v1, w_v2, gam_v0, gam_v1, gam_v2,
          tot_v, sem0, sem1, sem2):
        cid = lax.axis_index("c")
        sid = lax.axis_index("s")
        wid = sid * NC + cid
        base = wid * bpw
        iota16 = lax.iota(jnp.int32, L)
        perms = [jnp.bitwise_xor(iota16, k) for k in (8, 4, 2, 1)]
        total = jnp.zeros((L,), jnp.float32)

        bufs = [(x_v0, w_v0, gam_v0, sem0), (x_v1, w_v1, gam_v1, sem1),
                (x_v2, w_v2, gam_v2, sem2)]

        def issue(c):
            x_v, w_v, gam_v, sem = bufs[c % 3]
            r0 = base + c * CH
            pltpu.sync_copy(y_hbm.at[pl.ds(r0, CH)], idx_v.at[c])
            return (
                pltpu.async_copy(w_hbm.at[idx_v.at[c]], w_v, sem),
                pltpu.async_copy(g_hbm.at[idx_v.at[c]], gam_v, sem),
                pltpu.async_copy(x_hbm.at[pl.ds(r0, CH), :], x_v, sem),
            )

        def compute(c, total):
            x_v, w_v, gam_v, _ = bufs[c % 3]

            def grp_body(g, tot):
                gv16 = gam_v[pl.ds(g * L, L)]
                for j in range(L):
                    r = g * L + j
                    gsc = gv16[j]
                    acc = jnp.zeros((L,), jnp.float32)
                    for kk in range(D // L):
                        xv = x_v[r, pl.ds(kk * L, L)]
                        wv = w_v[r, pl.ds(kk * L, L)]
                        dv = xv - gsc * wv
                        acc = acc + dv * dv
                    nv = acc
                    for p in perms:
                        nv = nv + _shuffle(nv, p)
                    th = jnp.full((L,), gsc * THRESH_SCALE)
                    keep = nv > th * jnp.abs(th)
                    tot = tot + jnp.where(keep, acc, jnp.zeros((L,), jnp.float32))
                return tot

            return lax.fori_loop(0, ngrp, grp_body, total)

        depth = 2
        pending = [issue(c) for c in range(min(depth, nch))]
        for c in range(nch):
            if c + depth < nch:
                pending.append(issue(c + depth))
            for h in pending.pop(0):
                h.wait()
            total = compute(c, total)

        tot_v[...] = total
        pltpu.sync_copy(tot_v, out_hbm.at[wid])

    return k


def kernel(output_features, y_truth, fixed_weights, centers_gamma):
    B, D = output_features.shape
    C = fixed_weights.shape[0]
    k = _make_kernel(B, D, C)
    partials = k(
        output_features, y_truth, fixed_weights,
        centers_gamma.reshape(C),
    )
    return jnp.sum(partials) * (0.5 / B * LOSS_WEIGHT)


# speedup vs baseline: 1.0641x; 1.0641x over previous
import functools

import jax
import jax.numpy as jnp
from jax import lax
from jax.experimental import pallas as pl
from jax.experimental.pallas import tpu as pltpu
from jax.experimental.pallas import tpu_sc as plsc

LOSS_WEIGHT = 0.005
THRESH_SCALE = 0.05

NC = 2
NS = 16
L = 16
NW = NC * NS
CH = 128

_GDN = lax.GatherDimensionNumbers(
    offset_dims=(), collapsed_slice_dims=(0,), start_index_map=(0,)
)


def _shuffle(v, perm):
    return lax.gather(
        v,
        perm[:, None],
        _GDN,
        (1,),
        mode=lax.GatherScatterMode.PROMISE_IN_BOUNDS,
    )


def _make_kernel(B, D, C):
    bpw = B // NW
    nch = bpw // CH
    ngrp = CH // L

    mesh = plsc.VectorSubcoreMesh(core_axis_name="c", subcore_axis_name="s")

    @functools.partial(
        pl.kernel,
        mesh=mesh,
        out_type=jax.ShapeDtypeStruct((NW, L), jnp.float32),
        scratch_types=[
            pltpu.VMEM((nch * CH,), jnp.int32),
            pltpu.VMEM((CH, D), jnp.float32),
            pltpu.VMEM((CH, D), jnp.float32),
            pltpu.VMEM((CH, D), jnp.float32),
            pltpu.VMEM((CH, D), jnp.float32),
            pltpu.VMEM((CH, D), jnp.float32),
            pltpu.VMEM((CH, D), jnp.float32),
            pltpu.VMEM((CH,), jnp.float32),
            pltpu.VMEM((CH,), jnp.float32),
            pltpu.VMEM((CH,), jnp.float32),
            pltpu.VMEM((L,), jnp.float32),
            pltpu.SemaphoreType.DMA,
            pltpu.SemaphoreType.DMA,
            pltpu.SemaphoreType.DMA,
        ],
    )
    def k(x_hbm, y_hbm, w_hbm, g_hbm, out_hbm, idx_v,
          x_v0, x_v1, x_v2, w_v0, w_v1, w_v2, gam_v0, gam_v1, gam_v2,
          tot_v, sem0, sem1, sem2):
        cid = lax.axis_index("c")
        sid = lax.axis_index("s")
        wid = sid * NC + cid
        base = wid * bpw
        iota16 = lax.iota(jnp.int32, L)
        perms = [jnp.bitwise_xor(iota16, k) for k in (8, 4, 2, 1)]
        total = jnp.zeros((L,), jnp.float32)

        bufs = [(x_v0, w_v0, gam_v0, sem0), (x_v1, w_v1, gam_v1, sem1),
                (x_v2, w_v2, gam_v2, sem2)]

        pltpu.sync_copy(y_hbm.at[pl.ds(base, bpw)], idx_v)

        def issue(c):
            x_v, w_v, gam_v, sem = bufs[c % 3]
            r0 = base + c * CH
            idx_c = idx_v.at[pl.ds(c * CH, CH)]
            return (
                pltpu.async_copy(w_hbm.at[idx_c], w_v, sem),
                pltpu.async_copy(g_hbm.at[idx_c], gam_v, sem),
                pltpu.async_copy(x_hbm.at[pl.ds(r0, CH), :], x_v, sem),
            )

        def compute(c, total):
            x_v, w_v, gam_v, _ = bufs[c % 3]

            def grp_body(g, tot):
                gv16 = gam_v[pl.ds(g * L, L)]
                for j in range(L):
                    r = g * L + j
                    gsc = gv16[j]
                    acc = jnp.zeros((L,), jnp.float32)
                    for kk in range(D // L):
                        xv = x_v[r, pl.ds(kk * L, L)]
                        wv = w_v[r, pl.ds(kk * L, L)]
                        dv = xv - gsc * wv
                        acc = acc + dv * dv
                    nv = acc
                    for p in perms:
                        nv = nv + _shuffle(nv, p)
                    th = jnp.full((L,), gsc * THRESH_SCALE)
                    keep = nv > th * jnp.abs(th)
                    tot = tot + jnp.where(keep, acc, jnp.zeros((L,), jnp.float32))
                return tot

            return lax.fori_loop(0, ngrp, grp_body, total)

        depth = 2
        pending = [issue(c) for c in range(min(depth, nch))]
        for c in range(nch):
            if c + depth < nch:
                pending.append(issue(c + depth))
            for h in pending.pop(0):
                h.wait()
            total = compute(c, total)

        tot_v[...] = total
        pltpu.sync_copy(tot_v, out_hbm.at[wid])

    return k


def kernel(output_features, y_truth, fixed_weights, centers_gamma):
    B, D = output_features.shape
    C = fixed_weights.shape[0]
    k = _make_kernel(B, D, C)
    partials = k(
        output_features, y_truth, fixed_weights,
        centers_gamma.T.reshape(C),
    )
    return jnp.sum(partials) * (0.5 / B * LOSS_WEIGHT)


# speedup vs baseline: 1.1006x; 1.0343x over previous
import functools

import jax
import jax.numpy as jnp
from jax import lax
from jax.experimental import pallas as pl
from jax.experimental.pallas import tpu as pltpu
from jax.experimental.pallas import tpu_sc as plsc

LOSS_WEIGHT = 0.005
THRESH_SCALE = 0.05

NC = 2
NS = 16
L = 16
NW = NC * NS
CH = 128

_GDN = lax.GatherDimensionNumbers(
    offset_dims=(), collapsed_slice_dims=(0,), start_index_map=(0,)
)


def _shuffle(v, perm):
    return lax.gather(
        v,
        perm[:, None],
        _GDN,
        (1,),
        mode=lax.GatherScatterMode.PROMISE_IN_BOUNDS,
    )


def _make_kernel(B, D, C):
    bpw = B // NW
    nch = bpw // CH
    ngrp = CH // L

    mesh = plsc.VectorSubcoreMesh(core_axis_name="c", subcore_axis_name="s")

    @functools.partial(
        pl.kernel,
        mesh=mesh,
        out_type=jax.ShapeDtypeStruct((NW, L), jnp.float32),
        scratch_types=[
            pltpu.VMEM((nch, CH), jnp.int32),
            pltpu.VMEM((CH, D), jnp.float32),
            pltpu.VMEM((CH, D), jnp.float32),
            pltpu.VMEM((CH, D), jnp.float32),
            pltpu.VMEM((CH, D), jnp.float32),
            pltpu.VMEM((CH, D), jnp.float32),
            pltpu.VMEM((CH, D), jnp.float32),
            pltpu.VMEM((CH,), jnp.float32),
            pltpu.VMEM((CH,), jnp.float32),
            pltpu.VMEM((CH,), jnp.float32),
            pltpu.VMEM((L,), jnp.float32),
            pltpu.SemaphoreType.DMA,
            pltpu.SemaphoreType.DMA,
            pltpu.SemaphoreType.DMA,
        ],
    )
    def k(x_hbm, y_hbm, w_hbm, g_hbm, out_hbm, idx_v,
          x_v0, x_v1, x_v2, w_v0, w_v1, w_v2, gam_v0, gam_v1, gam_v2,
          tot_v, sem0, sem1, sem2):
        cid = lax.axis_index("c")
        sid = lax.axis_index("s")
        wid = sid * NC + cid
        base = wid * bpw
        iota16 = lax.iota(jnp.int32, L)
        perms = {k: jnp.bitwise_xor(iota16, k) for k in (8, 4, 2, 1)}
        total = jnp.zeros((L,), jnp.float32)

        bufs = [(x_v0, w_v0, gam_v0, sem0), (x_v1, w_v1, gam_v1, sem1),
                (x_v2, w_v2, gam_v2, sem2)]

        def issue(c):
            x_v, w_v, gam_v, sem = bufs[c % 3]
            r0 = base + c * CH
            pltpu.sync_copy(y_hbm.at[pl.ds(r0, CH)], idx_v.at[c])
            return (
                pltpu.async_copy(w_hbm.at[idx_v.at[c]], w_v, sem),
                pltpu.async_copy(g_hbm.at[idx_v.at[c]], gam_v, sem),
                pltpu.async_copy(x_hbm.at[pl.ds(r0, CH), :], x_v, sem),
            )

        def compute(c, total):
            x_v, w_v, gam_v, _ = bufs[c % 3]

            def grp_body(g, tot):
                gv16 = gam_v[pl.ds(g * L, L)]
                for j in range(L):
                    r = g * L + j
                    gsc = gv16[j]
                    acc = jnp.zeros((L,), jnp.float32)
                    for kk in range(D // L):
                        xv = x_v[r, pl.ds(kk * L, L)]
                        wv = w_v[r, pl.ds(kk * L, L)]
                        dv = xv - gsc * wv
                        acc = acc + dv * dv
                    nv = acc
                    for p in (perms[8], perms[4], perms[2], perms[1]):
                        nv = nv + _shuffle(nv, p)
                    th = jnp.full((L,), gsc * THRESH_SCALE)
                    keep = nv > th * jnp.abs(th)
                    tot = tot + jnp.where(keep, acc, jnp.zeros((L,), jnp.float32))
                return tot

            return lax.fori_loop(0, ngrp, grp_body, total)

        depth = 2
        pending = [issue(c) for c in range(min(depth, nch))]
        for c in range(nch):
            if c + depth < nch:
                pending.append(issue(c + depth))
            for h in pending.pop(0):
                h.wait()
            total = compute(c, total)

        tot_v[...] = total
        pltpu.sync_copy(tot_v, out_hbm.at[wid])

    return k


def kernel(output_features, y_truth, fixed_weights, centers_gamma):
    B, D = output_features.shape
    C = fixed_weights.shape[0]
    k = _make_kernel(B, D, C)
    partials = k(
        output_features, y_truth, fixed_weights,
        centers_gamma.T.reshape(C),
    )
    return jnp.sum(partials) * (0.5 / B * LOSS_WEIGHT)


# speedup vs baseline: 1.1180x; 1.0158x over previous
import functools

import jax
import jax.numpy as jnp
from jax import lax
from jax.experimental import pallas as pl
from jax.experimental.pallas import tpu as pltpu
from jax.experimental.pallas import tpu_sc as plsc

LOSS_WEIGHT = 0.005
THRESH_SCALE = 0.05

NC = 2
NS = 16
L = 16
NW = NC * NS
CH = 128

_GDN = lax.GatherDimensionNumbers(
    offset_dims=(), collapsed_slice_dims=(0,), start_index_map=(0,)
)


def _shuffle(v, perm):
    return lax.gather(
        v,
        perm[:, None],
        _GDN,
        (1,),
        mode=lax.GatherScatterMode.PROMISE_IN_BOUNDS,
    )


def _make_kernel(B, D, C):
    bpw = B // NW
    nch = bpw // CH
    ngrp = CH // L

    mesh = plsc.VectorSubcoreMesh(core_axis_name="c", subcore_axis_name="s")

    @functools.partial(
        pl.kernel,
        mesh=mesh,
        out_type=jax.ShapeDtypeStruct((NW, L), jnp.float32),
        scratch_types=[
            pltpu.VMEM((nch, CH), jnp.int32),
            pltpu.VMEM((CH, D), jnp.float32),
            pltpu.VMEM((CH, D), jnp.float32),
            pltpu.VMEM((CH, D), jnp.float32),
            pltpu.VMEM((CH, D), jnp.float32),
            pltpu.VMEM((CH, D), jnp.float32),
            pltpu.VMEM((CH, D), jnp.float32),
            pltpu.VMEM((CH,), jnp.float32),
            pltpu.VMEM((CH,), jnp.float32),
            pltpu.VMEM((CH,), jnp.float32),
            pltpu.VMEM((L,), jnp.float32),
            pltpu.SemaphoreType.DMA,
            pltpu.SemaphoreType.DMA,
            pltpu.SemaphoreType.DMA,
        ],
    )
    def k(x_hbm, y_hbm, w_hbm, g_hbm, out_hbm, idx_v,
          x_v0, x_v1, x_v2, w_v0, w_v1, w_v2, gam_v0, gam_v1, gam_v2,
          tot_v, sem0, sem1, sem2):
        cid = lax.axis_index("c")
        sid = lax.axis_index("s")
        wid = sid * NC + cid
        base = wid * bpw
        iota16 = lax.iota(jnp.int32, L)
        perms = {k: jnp.bitwise_xor(iota16, k) for k in (8, 4, 2, 1)}
        total = jnp.zeros((L,), jnp.float32)

        bufs = [(x_v0, w_v0, gam_v0, sem0), (x_v1, w_v1, gam_v1, sem1),
                (x_v2, w_v2, gam_v2, sem2)]

        def issue(c):
            x_v, w_v, gam_v, sem = bufs[c % 3]
            r0 = base + c * CH
            h_x = pltpu.async_copy(x_hbm.at[pl.ds(r0, CH), :], x_v, sem)
            pltpu.sync_copy(y_hbm.at[pl.ds(r0, CH)], idx_v.at[c])
            return (
                pltpu.async_copy(w_hbm.at[idx_v.at[c]], w_v, sem),
                pltpu.async_copy(g_hbm.at[idx_v.at[c]], gam_v, sem),
                h_x,
            )

        def compute(c, total):
            x_v, w_v, gam_v, _ = bufs[c % 3]

            def grp_body(g, tot):
                gv16 = gam_v[pl.ds(g * L, L)]
                for j in range(L):
                    r = g * L + j
                    gsc = gv16[j]
                    acc = jnp.zeros((L,), jnp.float32)
                    for kk in range(D // L):
                        xv = x_v[r, pl.ds(kk * L, L)]
                        wv = w_v[r, pl.ds(kk * L, L)]
                        dv = xv - gsc * wv
                        acc = acc + dv * dv
                    nv = acc
                    for p in (perms[8], perms[4], perms[2], perms[1]):
                        nv = nv + _shuffle(nv, p)
                    th = jnp.full((L,), gsc * THRESH_SCALE)
                    keep = nv > th * jnp.abs(th)
                    tot = tot + jnp.where(keep, acc, jnp.zeros((L,), jnp.float32))
                return tot

            return lax.fori_loop(0, ngrp, grp_body, total)

        depth = 2
        pending = [issue(c) for c in range(min(depth, nch))]
        for c in range(nch):
            if c + depth < nch:
                pending.append(issue(c + depth))
            for h in pending.pop(0):
                h.wait()
            total = compute(c, total)

        tot_v[...] = total
        pltpu.sync_copy(tot_v, out_hbm.at[wid])

    return k


def kernel(output_features, y_truth, fixed_weights, centers_gamma):
    B, D = output_features.shape
    C = fixed_weights.shape[0]
    k = _make_kernel(B, D, C)
    partials = k(
        output_features, y_truth, fixed_weights,
        centers_gamma.T.reshape(C),
    )
    return jnp.sum(partials) * (0.5 / B * LOSS_WEIGHT)
